# baseline (speedup 1.0000x reference)
import functools

import jax
import jax.numpy as jnp
from jax import lax
from jax.experimental import pallas as pl
from jax.experimental.pallas import tpu as pltpu
from jax.experimental.pallas import tpu_sc as plsc


def kernel(poi_idx, poi_embedding):
    B, H = poi_idx.shape
    V, D = poi_embedding.shape
    N = B * H
    info = plsc.get_sparse_core_info()
    NC, NS = info.num_cores, info.num_subcores
    NW = NC * NS
    C = 128
    J = N // (NW * C)
    assert N == NW * J * C
    NBUF = 10
    G = J // NBUF
    assert J == G * NBUF

    idx3 = poi_idx.reshape(NW, J, C)
    mesh = plsc.VectorSubcoreMesh(core_axis_name="c", subcore_axis_name="s")

    @functools.partial(
        pl.kernel,
        mesh=mesh,
        out_type=jax.ShapeDtypeStruct((N, D), jnp.float32),
        compiler_params=pltpu.CompilerParams(use_tc_tiling_on_sc=False),
        scratch_types=[
            pltpu.VMEM((J, C), jnp.int32),
            pltpu.VMEM((NBUF, C, D), jnp.float32),
            [pltpu.SemaphoreType.DMA] * NBUF,
            [pltpu.SemaphoreType.DMA] * NBUF,
        ],
    )
    def gather_kernel(idx_hbm, table_hbm, out_hbm, idx_v, rows_v, gsem, osem):
        wid = lax.axis_index("s") * NC + lax.axis_index("c")
        base = wid * (J * C)
        pltpu.sync_copy(idx_hbm.at[wid], idx_v)

        def start_gather(j, b):
            pltpu.async_copy(table_hbm.at[idx_v.at[j]], rows_v.at[b], gsem[b])

        def wait_gather(j, b):
            pltpu.make_async_copy(
                table_hbm.at[idx_v.at[j]], rows_v.at[b], gsem[b]
            ).wait()

        def start_out(j, b):
            pltpu.async_copy(
                rows_v.at[b], out_hbm.at[pl.ds(base + j * C, C)], osem[b]
            )

        def wait_out(j, b):
            pltpu.make_async_copy(
                rows_v.at[b], out_hbm.at[pl.ds(base + j * C, C)], osem[b]
            ).wait()

        for b in range(NBUF):
            start_gather(b, b)

        def group(g, carry):
            for b in range(NBUF):
                j = g * NBUF + b
                wait_gather(j, b)
                start_out(j, b)
            for b in range(NBUF):
                j = g * NBUF + b
                wait_out(j, b)
                start_gather(j + NBUF, b)
            return carry

        lax.fori_loop(0, G - 1, group, 0)

        for b in range(NBUF):
            j = (G - 1) * NBUF + b
            wait_gather(j, b)
            start_out(j, b)
        for b in range(NBUF):
            j = (G - 1) * NBUF + b
            wait_out(j, b)

    out = gather_kernel(idx3, poi_embedding)
    return out.reshape(B, H, D)


# speedup vs baseline: 1.0021x; 1.0021x over previous
import functools

import jax
import jax.numpy as jnp
from jax import lax
from jax.experimental import pallas as pl
from jax.experimental.pallas import tpu as pltpu
from jax.experimental.pallas import tpu_sc as plsc


def kernel(poi_idx, poi_embedding):
    B, H = poi_idx.shape
    V, D = poi_embedding.shape
    N = B * H
    info = plsc.get_sparse_core_info()
    NC, NS = info.num_cores, info.num_subcores
    NW = NC * NS
    C = 128
    J = N // (NW * C)
    assert N == NW * J * C
    NBUF = 10
    G = J // NBUF
    assert J == G * NBUF

    idx_flat = poi_idx.reshape(N)
    mesh = plsc.VectorSubcoreMesh(core_axis_name="c", subcore_axis_name="s")

    @functools.partial(
        pl.kernel,
        mesh=mesh,
        out_type=jax.ShapeDtypeStruct((N, D), jnp.float32),
        compiler_params=pltpu.CompilerParams(use_tc_tiling_on_sc=False),
        scratch_types=[
            pltpu.VMEM((J * C,), jnp.int32),
            pltpu.VMEM((NBUF, C, D), jnp.float32),
            [pltpu.SemaphoreType.DMA] * NBUF,
            [pltpu.SemaphoreType.DMA] * NBUF,
        ],
    )
    def gather_kernel(idx_hbm, table_hbm, out_hbm, idx_v, rows_v, gsem, osem):
        wid = lax.axis_index("s") * NC + lax.axis_index("c")
        base = wid * (J * C)
        pltpu.sync_copy(idx_hbm.at[pl.ds(base, J * C)], idx_v)

        def start_gather(j, b):
            pltpu.async_copy(
                table_hbm.at[idx_v.at[pl.ds(j * C, C)]], rows_v.at[b], gsem[b]
            )

        def wait_gather(j, b):
            pltpu.make_async_copy(
                table_hbm.at[idx_v.at[pl.ds(j * C, C)]], rows_v.at[b], gsem[b]
            ).wait()

        def start_out(j, b):
            pltpu.async_copy(
                rows_v.at[b], out_hbm.at[pl.ds(base + j * C, C)], osem[b]
            )

        def wait_out(j, b):
            pltpu.make_async_copy(
                rows_v.at[b], out_hbm.at[pl.ds(base + j * C, C)], osem[b]
            ).wait()

        for b in range(NBUF):
            start_gather(b, b)

        def group(g, carry):
            for b in range(NBUF):
                j = g * NBUF + b
                wait_gather(j, b)
                start_out(j, b)
            for b in range(NBUF):
                j = g * NBUF + b
                wait_out(j, b)
                start_gather(j + NBUF, b)
            return carry

        lax.fori_loop(0, G - 1, group, 0)

        for b in range(NBUF):
            j = (G - 1) * NBUF + b
            wait_gather(j, b)
            start_out(j, b)
        for b in range(NBUF):
            j = (G - 1) * NBUF + b
            wait_out(j, b)

    out = gather_kernel(idx_flat, poi_embedding)
    return out.reshape(B, H, D)
